# baseline (speedup 1.0000x reference)
import functools

import jax
import jax.numpy as jnp
from jax import lax
from jax.experimental import pallas as pl
from jax.experimental.pallas import tpu as pltpu
from jax.experimental.pallas import tpu_sc as plsc

B, N, EMB = 1024, 32, 64
NUM_BUCKETS = 201
NBLK = N * N
TVOL = NUM_BUCKETS * EMB

NC, NS = 2, 16
NW = NC * NS
BLK_W = NBLK // NW
HCHUNK = B // 2


def _body(dist_hbm, table_hbm, out_hbm,
          dv0, dv1, buf0, buf1, table_v,
          dist_sem, store_sem0, store_sem1):
    w = lax.axis_index("s") * NC + lax.axis_index("c")
    dv = (dv0, dv1)
    buf = (buf0, buf1)
    store_sem = (store_sem0, store_sem1)

    pltpu.sync_copy(table_hbm, table_v)

    def fire_dist(k, q):
        pltpu.async_copy(
            dist_hbm.at[pl.ds((w * BLK_W + k) * B, B)], dv[q], dist_sem)

    def wait_dist(q):
        pltpu.make_async_copy(
            dist_hbm.at[pl.ds(0, B)], dv[q], dist_sem).wait()

    def drain_store(h):
        pltpu.make_async_copy(
            buf[h], out_hbm.at[0, :, pl.ds(0, HCHUNK)], store_sem[h]).wait()

    def gather_chunk(q, h):
        def c_body(c, carry):
            v = dv[q][pl.ds(h * HCHUNK + c * 16, 16)]
            idx = jnp.clip(v, 0.0, float(NUM_BUCKETS - 1)).astype(jnp.int32)
            addr = idx * EMB
            for e in range(EMB):
                g = plsc.load_gather(table_v, [addr + e])
                buf[h][e, pl.ds(c * 16, 16)] = g
            return carry

        lax.fori_loop(0, HCHUNK // 16, c_body, 0)

    fire_dist(0, 0)

    def outer(kk, carry):
        for q in range(2):
            k = kk * 2 + q
            blk = w * BLK_W + k
            wait_dist(q)

            @pl.when(k + 1 < BLK_W)
            def _prefetch():
                fire_dist(k + 1, 1 - q)

            for h in range(2):
                @pl.when(k >= 1)
                def _drain():
                    drain_store(h)

                gather_chunk(q, h)
                pltpu.async_copy(
                    buf[h], out_hbm.at[blk, :, pl.ds(h * HCHUNK, HCHUNK)],
                    store_sem[h])
        return carry

    lax.fori_loop(0, BLK_W // 2, outer, 0)
    drain_store(0)
    drain_store(1)


def kernel(distance_matrix, table):
    dist_t = distance_matrix.transpose(1, 2, 0).reshape(NBLK * B)
    table_flat = table.reshape(TVOL)
    mesh = plsc.VectorSubcoreMesh(core_axis_name="c", subcore_axis_name="s")
    k = functools.partial(
        pl.kernel,
        out_type=jax.ShapeDtypeStruct((NBLK, EMB, B), jnp.float32),
        mesh=mesh,
        scratch_types=[
            pltpu.VMEM((B,), jnp.float32),
            pltpu.VMEM((B,), jnp.float32),
            pltpu.VMEM((EMB, HCHUNK), jnp.float32),
            pltpu.VMEM((EMB, HCHUNK), jnp.float32),
            pltpu.VMEM((TVOL,), jnp.float32),
            pltpu.SemaphoreType.DMA,
            pltpu.SemaphoreType.DMA,
            pltpu.SemaphoreType.DMA,
        ],
        compiler_params=pltpu.CompilerParams(
            use_tc_tiling_on_sc=True, needs_layout_passes=False),
    )(_body)
    out_t = k(dist_t, table_flat)
    return out_t.reshape(N, N, EMB, B).transpose(3, 0, 1, 2)


# speedup vs baseline: 1.7154x; 1.7154x over previous
import functools

import jax
import jax.numpy as jnp
from jax import lax
from jax.experimental import pallas as pl
from jax.experimental.pallas import tpu as pltpu
from jax.experimental.pallas import tpu_sc as plsc

B, N, EMB = 1024, 32, 64
NUM_BUCKETS = 201
NBLK = N * N
TVOL = NUM_BUCKETS * EMB

NC, NS = 2, 16
NW = NC * NS
BLK_W = NBLK // NW
HCHUNK = B // 2


def _body(dist_hbm, table_hbm, out_hbm,
          dv0, dv1, buf0, buf1, table_v,
          dist_sem, store_sem0, store_sem1):
    w = lax.axis_index("s") * NC + lax.axis_index("c")
    dv = (dv0, dv1)
    buf = (buf0, buf1)
    store_sem = (store_sem0, store_sem1)

    pltpu.sync_copy(table_hbm, table_v)

    def fire_dist(k, q):
        pltpu.async_copy(
            dist_hbm.at[pl.ds((w * BLK_W + k) * B, B)], dv[q], dist_sem)

    def wait_dist(q):
        pltpu.make_async_copy(
            dist_hbm.at[pl.ds(0, B)], dv[q], dist_sem).wait()

    def drain_store(h):
        pltpu.make_async_copy(
            buf[h], out_hbm.at[0, :, pl.ds(0, HCHUNK)], store_sem[h]).wait()

    def gather_chunk(q, h):
        def c_body(c, carry):
            v = dv[q][pl.ds(h * HCHUNK + c * 16, 16)]
            idx = jnp.clip(v, 0.0, float(NUM_BUCKETS - 1)).astype(jnp.int32)
            addr = idx * EMB
            for e0 in range(0, EMB, 16):
                gs = [plsc.load_gather(table_v, [addr + (e0 + j)])
                      for j in range(16)]
                for j in range(16):
                    buf[h][e0 + j, pl.ds(c * 16, 16)] = gs[j]
            return carry

        lax.fori_loop(0, HCHUNK // 16, c_body, 0)

    fire_dist(0, 0)

    def outer(kk, carry):
        for q in range(2):
            k = kk * 2 + q
            blk = w * BLK_W + k
            wait_dist(q)

            @pl.when(k + 1 < BLK_W)
            def _prefetch():
                fire_dist(k + 1, 1 - q)

            for h in range(2):
                @pl.when(k >= 1)
                def _drain():
                    drain_store(h)

                gather_chunk(q, h)
                pltpu.async_copy(
                    buf[h], out_hbm.at[blk, :, pl.ds(h * HCHUNK, HCHUNK)],
                    store_sem[h])
        return carry

    lax.fori_loop(0, BLK_W // 2, outer, 0)
    drain_store(0)
    drain_store(1)


def kernel(distance_matrix, table):
    dist_t = distance_matrix.transpose(1, 2, 0).reshape(NBLK * B)
    table_flat = table.reshape(TVOL)
    mesh = plsc.VectorSubcoreMesh(core_axis_name="c", subcore_axis_name="s")
    k = functools.partial(
        pl.kernel,
        out_type=jax.ShapeDtypeStruct((NBLK, EMB, B), jnp.float32),
        mesh=mesh,
        scratch_types=[
            pltpu.VMEM((B,), jnp.float32),
            pltpu.VMEM((B,), jnp.float32),
            pltpu.VMEM((EMB, HCHUNK), jnp.float32),
            pltpu.VMEM((EMB, HCHUNK), jnp.float32),
            pltpu.VMEM((TVOL,), jnp.float32),
            pltpu.SemaphoreType.DMA,
            pltpu.SemaphoreType.DMA,
            pltpu.SemaphoreType.DMA,
        ],
        compiler_params=pltpu.CompilerParams(
            use_tc_tiling_on_sc=True, needs_layout_passes=False),
    )(_body)
    out_t = k(dist_t, table_flat)
    return out_t.reshape(N, N, EMB, B).transpose(3, 0, 1, 2)


# speedup vs baseline: 8.3716x; 4.8801x over previous
import functools

import jax
import jax.numpy as jnp
from jax import lax
from jax.experimental import pallas as pl
from jax.experimental.pallas import tpu as pltpu
from jax.experimental.pallas import tpu_sc as plsc

B, N, EMB = 1024, 32, 64
NUM_BUCKETS = 201
NBLK = N * N
TVOL = NUM_BUCKETS * EMB

NC, NS = 2, 16
NW = NC * NS
BLK_W = NBLK // NW
HCHUNK = B // 2


def _body(dist_hbm, table_hbm, out_hbm,
          dv0, dv1, buf0, buf1, table_v,
          dist_sem, store_sem0, store_sem1):
    w = lax.axis_index("s") * NC + lax.axis_index("c")
    dv = (dv0, dv1)
    buf = (buf0, buf1)
    store_sem = (store_sem0, store_sem1)

    pltpu.sync_copy(table_hbm, table_v)

    def fire_dist(k, q):
        pltpu.async_copy(
            dist_hbm.at[pl.ds((w * BLK_W + k) * B, B)], dv[q], dist_sem)

    def wait_dist(q):
        pltpu.make_async_copy(
            dist_hbm.at[pl.ds(0, B)], dv[q], dist_sem).wait()

    def drain_store(h):
        pltpu.make_async_copy(
            buf[h], out_hbm.at[0, :, pl.ds(0, HCHUNK)], store_sem[h]).wait()

    def gather_chunk(q, h):
        def c_body(c, carry):
            v = dv[q][pl.ds(h * HCHUNK + c * 16, 16)]
            idx = jnp.clip(v, 0.0, float(NUM_BUCKETS - 1)).astype(jnp.int32)
            addr = idx
            for e0 in range(0, EMB, 16):
                gs = [plsc.load_gather(table_v,
                                       [addr + ((e0 + j) * NUM_BUCKETS)])
                      for j in range(16)]
                for j in range(16):
                    buf[h][e0 + j, pl.ds(c * 16, 16)] = gs[j]
            return carry

        lax.fori_loop(0, HCHUNK // 16, c_body, 0)

    fire_dist(0, 0)

    def outer(kk, carry):
        for q in range(2):
            k = kk * 2 + q
            blk = w * BLK_W + k
            wait_dist(q)

            @pl.when(k + 1 < BLK_W)
            def _prefetch():
                fire_dist(k + 1, 1 - q)

            for h in range(2):
                @pl.when(k >= 1)
                def _drain():
                    drain_store(h)

                gather_chunk(q, h)
                pltpu.async_copy(
                    buf[h], out_hbm.at[blk, :, pl.ds(h * HCHUNK, HCHUNK)],
                    store_sem[h])
        return carry

    lax.fori_loop(0, BLK_W // 2, outer, 0)
    drain_store(0)
    drain_store(1)


def kernel(distance_matrix, table):
    dist_t = distance_matrix.transpose(1, 2, 0).reshape(NBLK * B)
    table_flat = table.T.reshape(TVOL)
    mesh = plsc.VectorSubcoreMesh(core_axis_name="c", subcore_axis_name="s")
    k = functools.partial(
        pl.kernel,
        out_type=jax.ShapeDtypeStruct((NBLK, EMB, B), jnp.float32),
        mesh=mesh,
        scratch_types=[
            pltpu.VMEM((B,), jnp.float32),
            pltpu.VMEM((B,), jnp.float32),
            pltpu.VMEM((EMB, HCHUNK), jnp.float32),
            pltpu.VMEM((EMB, HCHUNK), jnp.float32),
            pltpu.VMEM((TVOL,), jnp.float32),
            pltpu.SemaphoreType.DMA,
            pltpu.SemaphoreType.DMA,
            pltpu.SemaphoreType.DMA,
        ],
        compiler_params=pltpu.CompilerParams(
            use_tc_tiling_on_sc=True, needs_layout_passes=False),
    )(_body)
    out_t = k(dist_t, table_flat)
    return out_t.reshape(N, N, EMB, B).transpose(3, 0, 1, 2)
